# baseline (speedup 1.0000x reference)
import functools

import jax
import jax.numpy as jnp
from jax import lax
from jax.experimental import pallas as pl
from jax.experimental.pallas import tpu as pltpu
from jax.experimental.pallas import tpu_sc as plsc

_N = 2_000_000
_LANES = 16
_ROWS = _N // _LANES
_NW = 32
_RPW = -(-_ROWS // _NW)
_CHUNK = 512
_FULL_CHUNKS = _RPW // _CHUNK
_NBINS = 15

_MESH = plsc.VectorSubcoreMesh(core_axis_name="c", subcore_axis_name="s")


@functools.partial(
    pl.kernel,
    out_type=jax.ShapeDtypeStruct((_NW * 48,), jnp.float32),
    mesh=_MESH,
    compiler_params=pltpu.CompilerParams(needs_layout_passes=False),
    scratch_types=[
        pltpu.VMEM((_CHUNK * 16,), jnp.float32),
        pltpu.VMEM((_CHUNK * 16,), jnp.int32),
        pltpu.VMEM((_CHUNK * 16,), jnp.int32),
        pltpu.VMEM((32,), jnp.float32),
        pltpu.VMEM((256,), jnp.float32),
        pltpu.VMEM((256,), jnp.float32),
        pltpu.VMEM((256,), jnp.float32),
        pltpu.VMEM((48,), jnp.float32),
    ],
)
def _sc_ece(conf_hbm, pred_hbm, lab_hbm, bnd_hbm, out_hbm,
            cbuf, pbuf, lbuf, bbuf, accc, accs, acca, sbuf):
    wid = lax.axis_index("s") * 2 + lax.axis_index("c")
    base_row = wid * _RPW
    my_end = jnp.minimum(base_row + _RPW, _ROWS)

    pltpu.sync_copy(bnd_hbm, bbuf)

    zv = jnp.zeros((16,), jnp.float32)
    for i in range(16):
        accc[pl.ds(i * 16, 16)] = zv
        accs[pl.ds(i * 16, 16)] = zv
        acca[pl.ds(i * 16, 16)] = zv

    lane16 = lax.iota(jnp.int32, 16) * 16
    ones = jnp.ones((16,), jnp.float32)

    def process_chunk(start_row, valid_from):
        pltpu.sync_copy(conf_hbm.at[pl.ds(start_row * 16, _CHUNK * 16)], cbuf)
        pltpu.sync_copy(pred_hbm.at[pl.ds(start_row * 16, _CHUNK * 16)], pbuf)
        pltpu.sync_copy(lab_hbm.at[pl.ds(start_row * 16, _CHUNK * 16)], lbuf)

        def row_body(i, carry):
            off = i * 16
            conf = cbuf[pl.ds(off, 16)]
            pred = pbuf[pl.ds(off, 16)]
            labv = lbuf[pl.ds(off, 16)]
            acc = jnp.where(pred == labv, 1.0, 0.0).astype(jnp.float32)
            b0 = (conf * 15.0 + 0.5).astype(jnp.int32)
            b0 = jnp.minimum(b0, 15)
            lo = plsc.load_gather(bbuf, [b0])
            b = b0 - jnp.where(conf <= lo, 1, 0)
            b = jnp.minimum(jnp.maximum(b, 0), _NBINS - 1)
            mask = conf > 0.0
            if valid_from is not None:
                mask = mask & ((start_row + i) >= valid_from)
            idx = lane16 + b
            plsc.addupdate_scatter(accc, [idx], ones, mask=mask)
            plsc.addupdate_scatter(accs, [idx], conf, mask=mask)
            plsc.addupdate_scatter(acca, [idx], acc, mask=mask)
            return carry

        lax.fori_loop(0, _CHUNK, row_body, 0)

    for k in range(_FULL_CHUNKS):
        process_chunk(base_row + k * _CHUNK, None)
    process_chunk(my_end - _CHUNK, base_row + _FULL_CHUNKS * _CHUNK)

    for q, ref in enumerate((accc, accs, acca)):
        tot = ref[pl.ds(0, 16)]
        for i in range(1, 16):
            tot = tot + ref[pl.ds(i * 16, 16)]
        sbuf[pl.ds(q * 16, 16)] = tot
    pltpu.sync_copy(sbuf, out_hbm.at[pl.ds(wid * 48, 48)])


def _tc_combine(p_ref, o_ref):
    p = p_ref[...]
    count = jnp.sum(p[:, 0:16], axis=0, keepdims=True)
    sumc = jnp.sum(p[:, 16:32], axis=0, keepdims=True)
    suma = jnp.sum(p[:, 32:48], axis=0, keepdims=True)
    safe = jnp.maximum(count, 1.0)
    prop = count / float(_N)
    term = jnp.where(count > 0.0,
                     jnp.abs(sumc / safe - suma / safe) * prop, 0.0)
    o_ref[...] = jnp.sum(term, keepdims=True)


def kernel(confidences, predictions, labels):
    bnd = jnp.linspace(0.0, 1.0, _NBINS + 1)
    bounds = jnp.concatenate(
        [bnd.astype(jnp.float32), jnp.full((16,), 2.0, jnp.float32)])
    partials = _sc_ece(confidences, predictions, labels, bounds)
    ece = pl.pallas_call(
        _tc_combine,
        out_shape=jax.ShapeDtypeStruct((1, 1), jnp.float32),
    )(partials.reshape(_NW, 48))
    return ece.reshape((1,))


# speedup vs baseline: 1.3788x; 1.3788x over previous
import functools

import jax
import jax.numpy as jnp
from jax import lax
from jax.experimental import pallas as pl
from jax.experimental.pallas import tpu as pltpu
from jax.experimental.pallas import tpu_sc as plsc

_N = 2_000_000
_LANES = 16
_ROWS = _N // _LANES
_NW = 32
_RPW = -(-_ROWS // _NW)
_CHUNK = 512
_FULL_CHUNKS = _RPW // _CHUNK
_NBINS = 15
_UNROLL = 8
_CBITS = 13
_ONE_ACC = (1 << _CBITS) + 1

_MESH = plsc.VectorSubcoreMesh(core_axis_name="c", subcore_axis_name="s")


@functools.partial(
    pl.kernel,
    out_type=jax.ShapeDtypeStruct((_NW * 48,), jnp.float32),
    mesh=_MESH,
    compiler_params=pltpu.CompilerParams(needs_layout_passes=False),
    scratch_types=[
        pltpu.VMEM((_CHUNK * 16,), jnp.float32),
        pltpu.VMEM((_CHUNK * 16,), jnp.float32),
        pltpu.VMEM((_CHUNK * 16,), jnp.int32),
        pltpu.VMEM((_CHUNK * 16,), jnp.int32),
        pltpu.VMEM((_CHUNK * 16,), jnp.int32),
        pltpu.VMEM((_CHUNK * 16,), jnp.int32),
        pltpu.VMEM((256,), jnp.int32),
        pltpu.VMEM((256,), jnp.float32),
        pltpu.VMEM((48,), jnp.float32),
        pltpu.SemaphoreType.DMA,
        pltpu.SemaphoreType.DMA,
    ],
)
def _sc_ece(conf_hbm, pred_hbm, lab_hbm, out_hbm,
            cbuf0, cbuf1, pbuf0, pbuf1, lbuf0, lbuf1,
            acci, accs, sbuf, sem0, sem1):
    bufs = ((cbuf0, pbuf0, lbuf0), (cbuf1, pbuf1, lbuf1))
    wid = lax.axis_index("s") * 2 + lax.axis_index("c")
    base_row = wid * _RPW
    my_end = jnp.minimum(base_row + _RPW, _ROWS)

    zi = jnp.zeros((16,), jnp.int32)
    zf = jnp.zeros((16,), jnp.float32)
    for i in range(16):
        acci[pl.ds(i * 16, 16)] = zi
        accs[pl.ds(i * 16, 16)] = zf

    lane16 = lax.iota(jnp.int32, 16) * 16
    delta = jnp.float32(1.0 / 15.0)
    sems = (sem0, sem1)

    starts = [base_row + k * _CHUNK for k in range(_FULL_CHUNKS)]
    starts.append(my_end - _CHUNK)
    valid_from = base_row + _FULL_CHUNKS * _CHUNK

    def issue(k):
        b = k % 2
        cb, pb, lb = bufs[b]
        sl = pl.ds(starts[k] * 16, _CHUNK * 16)
        return (pltpu.async_copy(conf_hbm.at[sl], cb, sems[b]),
                pltpu.async_copy(pred_hbm.at[sl], pb, sems[b]),
                pltpu.async_copy(lab_hbm.at[sl], lb, sems[b]))

    def compute(k):
        cb, pb, lb = bufs[k % 2]
        is_tail = k == _FULL_CHUNKS

        def row(off):
            conf = cb[pl.ds(off, 16)]
            pred = pb[pl.ds(off, 16)]
            labv = lb[pl.ds(off, 16)]
            packed = jnp.where(pred == labv, _ONE_ACC, 1)
            b0 = (conf * 15.0 + 0.5).astype(jnp.int32)
            bound = b0.astype(jnp.float32) * delta
            bi = b0 - jnp.where(conf <= bound, 1, 0)
            bi = jnp.maximum(bi, 0)
            mask = conf > 0.0
            return conf, packed, bi, mask

        def body(i, carry):
            base_off = i * (16 * _UNROLL)
            if is_tail:
                row0 = starts[k] + i * _UNROLL
            for u in range(_UNROLL):
                conf, packed, bi, mask = row(base_off + u * 16)
                if is_tail:
                    mask = mask & ((row0 + u) >= valid_from)
                idx = lane16 + bi
                plsc.addupdate_scatter(acci, [idx], packed, mask=mask)
                plsc.addupdate_scatter(accs, [idx], conf, mask=mask)
            return carry

        lax.fori_loop(0, _CHUNK // _UNROLL, body, 0)

    copies = issue(0)
    n_chunks = _FULL_CHUNKS + 1
    for k in range(n_chunks):
        for c in copies:
            c.wait()
        if k + 1 < n_chunks:
            copies = issue(k + 1)
        compute(k)

    toti = acci[pl.ds(0, 16)]
    totf = accs[pl.ds(0, 16)]
    for i in range(1, 16):
        toti = toti + acci[pl.ds(i * 16, 16)]
        totf = totf + accs[pl.ds(i * 16, 16)]
    count = (toti & ((1 << _CBITS) - 1)).astype(jnp.float32)
    sacc = (toti >> _CBITS).astype(jnp.float32)
    sbuf[pl.ds(0, 16)] = count
    sbuf[pl.ds(16, 16)] = totf
    sbuf[pl.ds(32, 16)] = sacc
    pltpu.sync_copy(sbuf, out_hbm.at[pl.ds(wid * 48, 48)])


def _tc_combine(p_ref, o_ref):
    p = p_ref[...]
    count = jnp.sum(p[:, 0:16], axis=0, keepdims=True)
    sumc = jnp.sum(p[:, 16:32], axis=0, keepdims=True)
    suma = jnp.sum(p[:, 32:48], axis=0, keepdims=True)
    safe = jnp.maximum(count, 1.0)
    prop = count / float(_N)
    term = jnp.where(count > 0.0,
                     jnp.abs(sumc / safe - suma / safe) * prop, 0.0)
    o_ref[...] = jnp.sum(term, keepdims=True)


def kernel(confidences, predictions, labels):
    partials = _sc_ece(confidences, predictions, labels)
    ece = pl.pallas_call(
        _tc_combine,
        out_shape=jax.ShapeDtypeStruct((1, 1), jnp.float32),
    )(partials.reshape(_NW, 48))
    return ece.reshape((1,))


# speedup vs baseline: 1.3971x; 1.0133x over previous
import functools

import jax
import jax.numpy as jnp
from jax import lax
from jax.experimental import pallas as pl
from jax.experimental.pallas import tpu as pltpu
from jax.experimental.pallas import tpu_sc as plsc

_N = 2_000_000
_LANES = 16
_ROWS = _N // _LANES
_NW = 32
_RPW = -(-_ROWS // _NW)
_CHUNK = 512
_FULL_CHUNKS = _RPW // _CHUNK
_NBINS = 15
_UNROLL = 8
_CBITS = 13
_ONE_ACC = (1 << _CBITS) + 1

_MESH = plsc.VectorSubcoreMesh(core_axis_name="c", subcore_axis_name="s")


@functools.partial(
    pl.kernel,
    out_type=jax.ShapeDtypeStruct((_NW * 720,), jnp.float32),
    mesh=_MESH,
    compiler_params=pltpu.CompilerParams(needs_layout_passes=False),
    scratch_types=[
        pltpu.VMEM((_CHUNK * 16,), jnp.float32),
        pltpu.VMEM((_CHUNK * 16,), jnp.float32),
        pltpu.VMEM((_CHUNK * 16,), jnp.int32),
        pltpu.VMEM((_CHUNK * 16,), jnp.int32),
        pltpu.VMEM((_CHUNK * 16,), jnp.int32),
        pltpu.VMEM((_CHUNK * 16,), jnp.int32),
        pltpu.VMEM((256,), jnp.int32),
        pltpu.VMEM((256,), jnp.float32),
        pltpu.VMEM((720,), jnp.float32),
        pltpu.SemaphoreType.DMA,
        pltpu.SemaphoreType.DMA,
    ],
)
def _sc_ece(conf_hbm, pred_hbm, lab_hbm, out_hbm,
            cbuf0, cbuf1, pbuf0, pbuf1, lbuf0, lbuf1,
            acci, accs, sbuf, sem0, sem1):
    bufs = ((cbuf0, pbuf0, lbuf0), (cbuf1, pbuf1, lbuf1))
    wid = lax.axis_index("s") * 2 + lax.axis_index("c")
    base_row = wid * _RPW
    my_end = jnp.minimum(base_row + _RPW, _ROWS)

    zi = jnp.zeros((16,), jnp.int32)
    zf = jnp.zeros((16,), jnp.float32)
    for i in range(16):
        acci[pl.ds(i * 16, 16)] = zi
        accs[pl.ds(i * 16, 16)] = zf

    lane = lax.iota(jnp.int32, 16)
    delta = jnp.float32(1.0 / 15.0)
    sems = (sem0, sem1)

    starts = [base_row + k * _CHUNK for k in range(_FULL_CHUNKS)]
    starts.append(my_end - _CHUNK)
    valid_from = base_row + _FULL_CHUNKS * _CHUNK

    def issue(k):
        b = k % 2
        cb, pb, lb = bufs[b]
        sl = pl.ds(starts[k] * 16, _CHUNK * 16)
        return (pltpu.async_copy(conf_hbm.at[sl], cb, sems[b]),
                pltpu.async_copy(pred_hbm.at[sl], pb, sems[b]),
                pltpu.async_copy(lab_hbm.at[sl], lb, sems[b]))

    def compute(k):
        cb, pb, lb = bufs[k % 2]
        is_tail = k == _FULL_CHUNKS

        def row(off):
            conf = cb[pl.ds(off, 16)]
            pred = pb[pl.ds(off, 16)]
            labv = lb[pl.ds(off, 16)]
            packed = jnp.where(pred == labv, _ONE_ACC, 1)
            b0 = (conf * 15.0 + 0.5).astype(jnp.int32)
            bound = b0.astype(jnp.float32) * delta
            bi = b0 - jnp.where(conf <= bound, 1, 0)
            bi = jnp.maximum(bi, 0)
            mask = conf > 0.0
            return conf, packed, bi, mask

        def body(i, carry):
            base_off = i * (16 * _UNROLL)
            if is_tail:
                row0 = starts[k] + i * _UNROLL
            for u in range(_UNROLL):
                conf, packed, bi, mask = row(base_off + u * 16)
                if is_tail:
                    mask = mask & ((row0 + u) >= valid_from)
                idx = bi * 16 + lane
                plsc.addupdate_scatter(acci, [idx], packed, mask=mask)
                plsc.addupdate_scatter(accs, [idx], conf, mask=mask)
            return carry

        lax.fori_loop(0, _CHUNK // _UNROLL, body, 0)

    copies = issue(0)
    n_chunks = _FULL_CHUNKS + 1
    for k in range(n_chunks):
        for c in copies:
            c.wait()
        if k + 1 < n_chunks:
            copies = issue(k + 1)
        compute(k)

    for bb in range(_NBINS):
        packed = acci[pl.ds(bb * 16, 16)]
        sbuf[pl.ds(bb * 16, 16)] = (
            packed & ((1 << _CBITS) - 1)).astype(jnp.float32)
        sbuf[pl.ds(240 + bb * 16, 16)] = accs[pl.ds(bb * 16, 16)]
        sbuf[pl.ds(480 + bb * 16, 16)] = (packed >> _CBITS).astype(jnp.float32)
    pltpu.sync_copy(sbuf, out_hbm.at[pl.ds(wid * 720, 720)])


def _tc_combine(p_ref, o_ref):
    q = jnp.sum(p_ref[...], axis=0)
    s = jnp.sum(q, axis=1)
    count = s[0:15]
    sumc = s[15:30]
    suma = s[30:45]
    safe = jnp.maximum(count, 1.0)
    prop = count / float(_N)
    term = jnp.where(count > 0.0,
                     jnp.abs(sumc / safe - suma / safe) * prop, 0.0)
    o_ref[...] = jnp.sum(term).reshape(1, 1)


def kernel(confidences, predictions, labels):
    partials = _sc_ece(confidences, predictions, labels)
    ece = pl.pallas_call(
        _tc_combine,
        out_shape=jax.ShapeDtypeStruct((1, 1), jnp.float32),
    )(partials.reshape(_NW, 45, 16))
    return ece.reshape((1,))


# speedup vs baseline: 3.4599x; 2.4765x over previous
import functools

import jax
import jax.numpy as jnp
from jax import lax
from jax.experimental import pallas as pl
from jax.experimental.pallas import tpu as pltpu
from jax.experimental.pallas import tpu_sc as plsc

_N = 2_000_000
_LANES = 16
_ROWS = _N // _LANES
_NW = 32
_RPW = -(-_ROWS // _NW)
_CHUNK = 512
_FULL_CHUNKS = _RPW // _CHUNK
_NBINS = 15
_UNROLL = 8
_CBITS = 13
_ONE_ACC = (1 << _CBITS) + 1

_MESH = plsc.VectorSubcoreMesh(core_axis_name="c", subcore_axis_name="s")


@functools.partial(
    pl.kernel,
    out_type=jax.ShapeDtypeStruct((_NW * 720,), jnp.float32),
    mesh=_MESH,
    compiler_params=pltpu.CompilerParams(needs_layout_passes=False),
    scratch_types=[
        pltpu.VMEM((_CHUNK * 16,), jnp.float32),
        pltpu.VMEM((_CHUNK * 16,), jnp.float32),
        pltpu.VMEM((_CHUNK * 16,), jnp.int32),
        pltpu.VMEM((_CHUNK * 16,), jnp.int32),
        pltpu.VMEM((_CHUNK * 16,), jnp.int32),
        pltpu.VMEM((_CHUNK * 16,), jnp.int32),
        pltpu.VMEM((256,), jnp.int32),
        pltpu.VMEM((256,), jnp.float32),
        pltpu.VMEM((720,), jnp.float32),
        pltpu.SemaphoreType.DMA,
        pltpu.SemaphoreType.DMA,
    ],
)
def _sc_ece(conf_hbm, pred_hbm, lab_hbm, out_hbm,
            cbuf0, cbuf1, pbuf0, pbuf1, lbuf0, lbuf1,
            acci, accs, sbuf, sem0, sem1):
    bufs = ((cbuf0, pbuf0, lbuf0), (cbuf1, pbuf1, lbuf1))
    wid = lax.axis_index("s") * 2 + lax.axis_index("c")
    base_row = wid * _RPW
    my_end = jnp.minimum(base_row + _RPW, _ROWS)

    zi = jnp.zeros((16,), jnp.int32)
    zf = jnp.zeros((16,), jnp.float32)
    for i in range(16):
        acci[pl.ds(i * 16, 16)] = zi
        accs[pl.ds(i * 16, 16)] = zf

    lane = lax.iota(jnp.int32, 16)
    delta = jnp.float32(1.0 / 15.0)
    sems = (sem0, sem1)

    starts = [base_row + k * _CHUNK for k in range(_FULL_CHUNKS)]
    starts.append(my_end - _CHUNK)
    valid_from = base_row + _FULL_CHUNKS * _CHUNK

    def issue(k):
        b = k % 2
        cb, pb, lb = bufs[b]
        sl = pl.ds(starts[k] * 16, _CHUNK * 16)
        return (pltpu.async_copy(conf_hbm.at[sl], cb, sems[b]),
                pltpu.async_copy(pred_hbm.at[sl], pb, sems[b]),
                pltpu.async_copy(lab_hbm.at[sl], lb, sems[b]))

    def compute(k):
        cb, pb, lb = bufs[k % 2]
        is_tail = k == _FULL_CHUNKS

        def row(off):
            conf = cb[pl.ds(off, 16)]
            pred = pb[pl.ds(off, 16)]
            labv = lb[pl.ds(off, 16)]
            packed = jnp.where(pred == labv, _ONE_ACC, 1)
            b0 = (conf * 15.0 + 0.5).astype(jnp.int32)
            bound = b0.astype(jnp.float32) * delta
            bi = b0 - jnp.where(conf <= bound, 1, 0)
            bi = jnp.maximum(bi, 0)
            mask = conf > 0.0
            return conf, packed, bi, mask

        @plsc.parallel_loop(0, _CHUNK, unroll=_UNROLL)
        def body(i):
            conf, packed, bi, mask = row(i * 16)
            if is_tail:
                mask = mask & ((starts[k] + i) >= valid_from)
            idx = bi * 16 + lane
            plsc.addupdate_scatter(acci, [idx], packed, mask=mask)
            plsc.addupdate_scatter(accs, [idx], conf, mask=mask)

    copies = issue(0)
    n_chunks = _FULL_CHUNKS + 1
    for k in range(n_chunks):
        for c in copies:
            c.wait()
        if k + 1 < n_chunks:
            copies = issue(k + 1)
        compute(k)

    for bb in range(_NBINS):
        packed = acci[pl.ds(bb * 16, 16)]
        sbuf[pl.ds(bb * 16, 16)] = (
            packed & ((1 << _CBITS) - 1)).astype(jnp.float32)
        sbuf[pl.ds(240 + bb * 16, 16)] = accs[pl.ds(bb * 16, 16)]
        sbuf[pl.ds(480 + bb * 16, 16)] = (packed >> _CBITS).astype(jnp.float32)
    pltpu.sync_copy(sbuf, out_hbm.at[pl.ds(wid * 720, 720)])


def _tc_combine(p_ref, o_ref):
    q = jnp.sum(p_ref[...], axis=0)
    s = jnp.sum(q, axis=1)
    count = s[0:15]
    sumc = s[15:30]
    suma = s[30:45]
    safe = jnp.maximum(count, 1.0)
    prop = count / float(_N)
    term = jnp.where(count > 0.0,
                     jnp.abs(sumc / safe - suma / safe) * prop, 0.0)
    o_ref[...] = jnp.sum(term).reshape(1, 1)


def kernel(confidences, predictions, labels):
    partials = _sc_ece(confidences, predictions, labels)
    ece = pl.pallas_call(
        _tc_combine,
        out_shape=jax.ShapeDtypeStruct((1, 1), jnp.float32),
    )(partials.reshape(_NW, 45, 16))
    return ece.reshape((1,))


# speedup vs baseline: 3.8705x; 1.1187x over previous
import functools

import jax
import jax.numpy as jnp
from jax import lax
from jax.experimental import pallas as pl
from jax.experimental.pallas import tpu as pltpu
from jax.experimental.pallas import tpu_sc as plsc

_N = 2_000_000
_LANES = 16
_ROWS = _N // _LANES
_NW = 32
_RPW = -(-_ROWS // _NW)
_CHUNK = 1024
_FULL_CHUNKS = _RPW // _CHUNK
_NBINS = 15
_UNROLL = 8
_CBITS = 13
_ONE_ACC = (1 << _CBITS) + 1

_MESH = plsc.VectorSubcoreMesh(core_axis_name="c", subcore_axis_name="s")


@functools.partial(
    pl.kernel,
    out_type=jax.ShapeDtypeStruct((_NW, 45, 16), jnp.float32),
    mesh=_MESH,
    compiler_params=pltpu.CompilerParams(needs_layout_passes=False),
    scratch_types=[
        pltpu.VMEM((_CHUNK * 16,), jnp.float32),
        pltpu.VMEM((_CHUNK * 16,), jnp.float32),
        pltpu.VMEM((_CHUNK * 16,), jnp.int32),
        pltpu.VMEM((_CHUNK * 16,), jnp.int32),
        pltpu.VMEM((_CHUNK * 16,), jnp.int32),
        pltpu.VMEM((_CHUNK * 16,), jnp.int32),
        pltpu.VMEM((272,), jnp.int32),
        pltpu.VMEM((272,), jnp.float32),
        pltpu.VMEM((45, 16), jnp.float32),
        pltpu.SemaphoreType.DMA,
        pltpu.SemaphoreType.DMA,
    ],
)
def _sc_ece(conf_hbm, pred_hbm, lab_hbm, out_hbm,
            cbuf0, cbuf1, pbuf0, pbuf1, lbuf0, lbuf1,
            acci, accs, sbuf, sem0, sem1):
    bufs = ((cbuf0, pbuf0, lbuf0), (cbuf1, pbuf1, lbuf1))
    wid = lax.axis_index("s") * 2 + lax.axis_index("c")
    base_row = wid * _RPW
    my_end = jnp.minimum(base_row + _RPW, _ROWS)

    zi = jnp.zeros((16,), jnp.int32)
    zf = jnp.zeros((16,), jnp.float32)
    for i in range(17):
        acci[pl.ds(i * 16, 16)] = zi
        accs[pl.ds(i * 16, 16)] = zf

    lane = lax.iota(jnp.int32, 16)
    lane16 = lane + 16
    delta = jnp.float32(1.0 / 15.0)
    sems = (sem0, sem1)

    starts = [base_row + k * _CHUNK for k in range(_FULL_CHUNKS)]
    starts.append(my_end - _CHUNK)
    valid_from = base_row + _FULL_CHUNKS * _CHUNK

    def issue(k):
        b = k % 2
        cb, pb, lb = bufs[b]
        sl = pl.ds(starts[k] * 16, _CHUNK * 16)
        return (pltpu.async_copy(conf_hbm.at[sl], cb, sems[b]),
                pltpu.async_copy(pred_hbm.at[sl], pb, sems[b]),
                pltpu.async_copy(lab_hbm.at[sl], lb, sems[b]))

    def compute(k):
        cb, pb, lb = bufs[k % 2]
        is_tail = k == _FULL_CHUNKS

        @plsc.parallel_loop(0, _CHUNK, unroll=_UNROLL)
        def body(i):
            off = i * 16
            conf = cb[pl.ds(off, 16)]
            pred = pb[pl.ds(off, 16)]
            labv = lb[pl.ds(off, 16)]
            packed = jnp.where(pred == labv, _ONE_ACC, 1)
            b0 = (conf * 15.0 + 0.5).astype(jnp.int32)
            bound = b0.astype(jnp.float32) * delta
            adj = jnp.where(conf <= bound, lane, lane16)
            idx = b0 * 16 + adj
            if is_tail:
                ok = jnp.broadcast_to((starts[k] + i) >= valid_from, (16,))
                idx = jnp.where(ok, idx, lane)
            plsc.addupdate_scatter(acci, [idx], packed)
            plsc.addupdate_scatter(accs, [idx], conf)

    copies = issue(0)
    n_chunks = _FULL_CHUNKS + 1
    for k in range(n_chunks):
        for c in copies:
            c.wait()
        if k + 1 < n_chunks:
            copies = issue(k + 1)
        compute(k)

    for bb in range(_NBINS):
        packed = acci[pl.ds((bb + 1) * 16, 16)]
        sbuf[bb] = (packed & ((1 << _CBITS) - 1)).astype(jnp.float32)
        sbuf[15 + bb] = accs[pl.ds((bb + 1) * 16, 16)]
        sbuf[30 + bb] = (packed >> _CBITS).astype(jnp.float32)
    pltpu.sync_copy(sbuf, out_hbm.at[wid])


def _tc_combine(p_ref, o_ref):
    q = jnp.sum(p_ref[...], axis=0)
    s = jnp.sum(q, axis=1)
    count = s[0:15]
    sumc = s[15:30]
    suma = s[30:45]
    safe = jnp.maximum(count, 1.0)
    prop = count / float(_N)
    term = jnp.where(count > 0.0,
                     jnp.abs(sumc / safe - suma / safe) * prop, 0.0)
    o_ref[...] = jnp.sum(term).reshape(1, 1)


def kernel(confidences, predictions, labels):
    partials = _sc_ece(confidences, predictions, labels)
    ece = pl.pallas_call(
        _tc_combine,
        out_shape=jax.ShapeDtypeStruct((1, 1), jnp.float32),
    )(partials)
    return ece.reshape((1,))
